# Initial kernel scaffold; baseline (speedup 1.0000x reference)
import jax
import jax.numpy as jnp
from jax.experimental import pallas as pl


def kernel(mesh_ids, sources):
    raise NotImplementedError("write your pallas kernel here")


# speedup vs baseline: 8.8796x; 8.8796x over previous
import functools

import jax
import jax.numpy as jnp
from jax import lax
from jax.experimental import pallas as pl
from jax.experimental.pallas import tpu as pltpu
from jax.experimental.pallas import tpu_sc as plsc

N = 320000
D = 128
M = 10000
NC = 2
NS = 16
NW = NC * NS
ROWS_PER_TILE = N // NW
CH = 80
NCH = ROWS_PER_TILE // CH
MPT = 624

_mesh = plsc.VectorSubcoreMesh(core_axis_name="c", subcore_axis_name="s")


@functools.partial(
    pl.kernel,
    out_type=(
        jax.ShapeDtypeStruct((NC, M, D), jnp.float32),
        jax.ShapeDtypeStruct((NW * M,), jnp.int32),
    ),
    mesh=_mesh,
    scratch_types=[
        pltpu.VMEM((CH,), jnp.int32),
        pltpu.VMEM((CH,), jnp.int32),
        pltpu.VMEM((CH, D), jnp.float32),
        pltpu.VMEM((CH, D), jnp.float32),
        pltpu.VMEM((M + 16,), jnp.int32),
        pltpu.VMEM_SHARED((M, D), jnp.float32),
        pltpu.SemaphoreType.DMA,
        pltpu.SemaphoreType.DMA,
        pltpu.SemaphoreType.DMA,
        pltpu.SemaphoreType.DMA,
        pltpu.SemaphoreType.DMA,
        pltpu.SemaphoreType.DMA,
    ],
)
def _segment_sum_sc(ids_hbm, src_hbm, zacc_hbm,
                    sums_out, cnts_out,
                    idx_a, idx_b, buf_a, buf_b, cnt_v, acc_sh,
                    sem_ai, sem_ar, sem_bi, sem_br, sem_as, sem_bs):
  cid = lax.axis_index("c")
  sid = lax.axis_index("s")
  wid = cid * NS + sid
  base = wid * ROWS_PER_TILE
  mrow = sid * MPT
  tail = NS * MPT

  pltpu.sync_copy(zacc_hbm.at[pl.ds(mrow, MPT)], acc_sh.at[pl.ds(mrow, MPT)])

  @pl.when(sid == 0)
  def _():
    pltpu.sync_copy(zacc_hbm.at[pl.ds(tail, M - tail)],
                    acc_sh.at[pl.ds(tail, M - tail)])

  zero16 = jnp.zeros((16,), jnp.int32)

  def zfill(i, carry):
    cnt_v[pl.ds(i * 16, 16)] = zero16
    return carry

  lax.fori_loop(0, (M + 16) // 16, zfill, None)
  plsc.subcore_barrier()

  def start(ch, idx_v, buf_v, sem_i, sem_r):
    off = base + ch * CH
    pltpu.async_copy(ids_hbm.at[pl.ds(off, CH)], idx_v, sem_i)
    pltpu.async_copy(src_hbm.at[pl.ds(off, CH)], buf_v, sem_r)

  def wait(idx_v, buf_v, sem_i, sem_r):
    pltpu.make_async_copy(ids_hbm.at[pl.ds(0, CH)], idx_v, sem_i).wait()
    pltpu.make_async_copy(src_hbm.at[pl.ds(0, CH)], buf_v, sem_r).wait()

  onehot0 = jnp.where(lax.iota(jnp.int32, 16) == 0, 1, 0)

  def count_rows(idx_v):
    for j in range(CH // 16):
      v = idx_v[pl.ds(j * 16, 16)]
      for l in range(16):
        s = v[l]
        w = cnt_v[pl.ds(s, 16)]
        cnt_v[pl.ds(s, 16)] = w + onehot0

  def process(idx_v, buf_v, sem_s):
    cp = pltpu.async_copy(buf_v, acc_sh.at[idx_v], sem_s, add=True)
    count_rows(idx_v)
    cp.wait()

  start(0, idx_a, buf_a, sem_ai, sem_ar)
  start(1, idx_b, buf_b, sem_bi, sem_br)

  def body(k, carry):
    wait(idx_a, buf_a, sem_ai, sem_ar)
    process(idx_a, buf_a, sem_as)
    start(2 * k + 2, idx_a, buf_a, sem_ai, sem_ar)
    wait(idx_b, buf_b, sem_bi, sem_br)
    process(idx_b, buf_b, sem_bs)

    @pl.when(k < NCH // 2 - 1)
    def _():
      start(2 * k + 3, idx_b, buf_b, sem_bi, sem_br)

    return carry

  lax.fori_loop(0, NCH // 2, body, None)
  wait(idx_a, buf_a, sem_ai, sem_ar)
  process(idx_a, buf_a, sem_as)

  plsc.subcore_barrier()
  pltpu.sync_copy(acc_sh.at[pl.ds(mrow, MPT)],
                  sums_out.at[cid, pl.ds(mrow, MPT)])
  pltpu.sync_copy(cnt_v.at[pl.ds(0, M)], cnts_out.at[pl.ds(wid * M, M)])

  @pl.when(sid == 0)
  def _():
    pltpu.sync_copy(acc_sh.at[pl.ds(tail, M - tail)],
                    sums_out.at[cid, pl.ds(tail, M - tail)])


def _combine_body(s_ref, c_ref, o_ref):
  s = s_ref[0] + s_ref[1]
  c = jnp.sum(c_ref[...], axis=1, keepdims=True).astype(jnp.float32)
  o_ref[...] = s / jnp.maximum(c, 1.0)


_BM = 2000

_combine = pl.pallas_call(
    _combine_body,
    grid=(M // _BM,),
    in_specs=[
        pl.BlockSpec((NC, _BM, D), lambda i: (0, i, 0)),
        pl.BlockSpec((_BM, NW), lambda i: (i, 0)),
    ],
    out_specs=pl.BlockSpec((_BM, D), lambda i: (i, 0)),
    out_shape=jax.ShapeDtypeStruct((M, D), jnp.float32),
)


@jax.jit
def kernel(mesh_ids, sources):
  zacc = jnp.zeros((M, D), jnp.float32)
  sums, cnts = _segment_sum_sc(mesh_ids, sources, zacc)
  cnts_t = cnts.reshape(NW, M).T
  return _combine(sums, cnts_t)


# speedup vs baseline: 9.5110x; 1.0711x over previous
import functools

import jax
import jax.numpy as jnp
from jax import lax
from jax.experimental import pallas as pl
from jax.experimental.pallas import tpu as pltpu
from jax.experimental.pallas import tpu_sc as plsc

N = 320000
D = 128
M = 10000
NC = 2
NS = 16
NW = NC * NS
ROWS_PER_TILE = N // NW
CH = 80
NCH = ROWS_PER_TILE // CH
Q = 3
G = NCH // Q
MPT = 624

_mesh = plsc.VectorSubcoreMesh(core_axis_name="c", subcore_axis_name="s")


@functools.partial(
    pl.kernel,
    out_type=(
        jax.ShapeDtypeStruct((NC, M, D), jnp.float32),
        jax.ShapeDtypeStruct((NW * M,), jnp.int32),
    ),
    mesh=_mesh,
    scratch_types=[
        [pltpu.VMEM((CH,), jnp.int32) for _ in range(Q)],
        [pltpu.VMEM((CH, D), jnp.float32) for _ in range(Q)],
        pltpu.VMEM((M + 16,), jnp.int32),
        pltpu.VMEM_SHARED((M, D), jnp.float32),
        [pltpu.SemaphoreType.DMA for _ in range(Q)],
        [pltpu.SemaphoreType.DMA for _ in range(Q)],
        [pltpu.SemaphoreType.DMA for _ in range(Q)],
    ],
)
def _segment_sum_sc(ids_hbm, src_hbm, zacc_hbm,
                    sums_out, cnts_out,
                    idx_ring, buf_ring, cnt_v, acc_sh, sem_i, sem_r, sem_s):
  cid = lax.axis_index("c")
  sid = lax.axis_index("s")
  wid = cid * NS + sid
  base = wid * ROWS_PER_TILE
  mrow = sid * MPT
  tail = NS * MPT

  pltpu.sync_copy(zacc_hbm.at[pl.ds(mrow, MPT)], acc_sh.at[pl.ds(mrow, MPT)])

  @pl.when(sid == 0)
  def _():
    pltpu.sync_copy(zacc_hbm.at[pl.ds(tail, M - tail)],
                    acc_sh.at[pl.ds(tail, M - tail)])

  zero16 = jnp.zeros((16,), jnp.int32)

  def zfill(i, carry):
    cnt_v[pl.ds(i * 16, 16)] = zero16
    return carry

  lax.fori_loop(0, (M + 16) // 16, zfill, None)
  plsc.subcore_barrier()

  onehot0 = jnp.where(lax.iota(jnp.int32, 16) == 0, 1, 0)

  def count_rows(idx_v):
    for j in range(CH // 16):
      v = idx_v[pl.ds(j * 16, 16)]
      for l in range(16):
        s = v[l]
        w = cnt_v[pl.ds(s, 16)]
        cnt_v[pl.ds(s, 16)] = w + onehot0

  def fire_chunk(q, ch):
    off = base + ch * CH
    pltpu.async_copy(ids_hbm.at[pl.ds(off, CH)], idx_ring[q], sem_i[q])
    pltpu.async_copy(src_hbm.at[pl.ds(off, CH)], buf_ring[q], sem_r[q])

  def wait_chunk(q):
    pltpu.make_async_copy(ids_hbm.at[pl.ds(0, CH)],
                          idx_ring[q], sem_i[q]).wait()
    pltpu.make_async_copy(src_hbm.at[pl.ds(0, CH)],
                          buf_ring[q], sem_r[q]).wait()

  def fire_scatter(q):
    pltpu.async_copy(buf_ring[q], acc_sh.at[idx_ring[q]], sem_s[q], add=True)

  def wait_scatter(q):
    pltpu.make_async_copy(buf_ring[q], acc_sh.at[idx_ring[q]],
                          sem_s[q]).wait()

  def process(q):
    wait_chunk(q)
    cp = pltpu.async_copy(buf_ring[q], acc_sh.at[idx_ring[q]], sem_s[q],
                          add=True)
    count_rows(idx_ring[q])
    cp.wait()

  for q in range(Q):
    fire_chunk(q, q)

  def body(g, carry):
    for q in range(Q):
      process(q)
      nxt = Q * g + q + Q

      @pl.when(nxt < NCH)
      def _():
        fire_chunk(q, nxt)

    return carry

  lax.fori_loop(0, G, body, None)
  for q in range(NCH - Q * G):
    process(q)

  plsc.subcore_barrier()
  pltpu.sync_copy(acc_sh.at[pl.ds(mrow, MPT)],
                  sums_out.at[cid, pl.ds(mrow, MPT)])
  pltpu.sync_copy(cnt_v.at[pl.ds(0, M)], cnts_out.at[pl.ds(wid * M, M)])

  @pl.when(sid == 0)
  def _():
    pltpu.sync_copy(acc_sh.at[pl.ds(tail, M - tail)],
                    sums_out.at[cid, pl.ds(tail, M - tail)])


def _combine_body(s_ref, c_ref, o_ref):
  s = s_ref[0] + s_ref[1]
  c = jnp.sum(c_ref[...], axis=1, keepdims=True).astype(jnp.float32)
  o_ref[...] = s / jnp.maximum(c, 1.0)


_BM = 2000

_combine = pl.pallas_call(
    _combine_body,
    grid=(M // _BM,),
    in_specs=[
        pl.BlockSpec((NC, _BM, D), lambda i: (0, i, 0)),
        pl.BlockSpec((_BM, NW), lambda i: (i, 0)),
    ],
    out_specs=pl.BlockSpec((_BM, D), lambda i: (i, 0)),
    out_shape=jax.ShapeDtypeStruct((M, D), jnp.float32),
)


@jax.jit
def kernel(mesh_ids, sources):
  zacc = jnp.zeros((M, D), jnp.float32)
  sums, cnts = _segment_sum_sc(mesh_ids, sources, zacc)
  cnts_t = cnts.reshape(NW, M).T
  return _combine(sums, cnts_t)


# speedup vs baseline: 9.5634x; 1.0055x over previous
import functools

import jax
import jax.numpy as jnp
from jax import lax
from jax.experimental import pallas as pl
from jax.experimental.pallas import tpu as pltpu
from jax.experimental.pallas import tpu_sc as plsc

N = 320000
D = 128
M = 10000
NC = 2
NS = 16
NW = NC * NS
ROWS_PER_TILE = N // NW
CH = 80
NCH = ROWS_PER_TILE // CH
Q = 3
G = NCH // Q
MPT = 624

_mesh = plsc.VectorSubcoreMesh(core_axis_name="c", subcore_axis_name="s")


@functools.partial(
    pl.kernel,
    out_type=(
        jax.ShapeDtypeStruct((NC, M, D), jnp.float32),
        jax.ShapeDtypeStruct((NW * M,), jnp.int32),
    ),
    mesh=_mesh,
    scratch_types=[
        [pltpu.VMEM((CH,), jnp.int32) for _ in range(Q)],
        [pltpu.VMEM((CH, D), jnp.float32) for _ in range(Q)],
        pltpu.VMEM((M + 16,), jnp.int32),
        pltpu.VMEM_SHARED((M, D), jnp.float32),
        [pltpu.SemaphoreType.DMA for _ in range(Q)],
        [pltpu.SemaphoreType.DMA for _ in range(Q)],
        [pltpu.SemaphoreType.DMA for _ in range(Q)],
    ],
)
def _segment_sum_sc(ids_hbm, src_hbm, zacc_hbm,
                    sums_out, cnts_out,
                    idx_ring, buf_ring, cnt_v, acc_sh, sem_i, sem_r, sem_s):
  cid = lax.axis_index("c")
  sid = lax.axis_index("s")
  wid = cid * NS + sid
  base = wid * ROWS_PER_TILE
  mrow = sid * MPT
  tail = NS * MPT

  def fire_chunk(q, ch):
    off = base + ch * CH
    pltpu.async_copy(ids_hbm.at[pl.ds(off, CH)], idx_ring[q], sem_i[q])
    pltpu.async_copy(src_hbm.at[pl.ds(off, CH)], buf_ring[q], sem_r[q])

  for q in range(Q):
    fire_chunk(q, q)

  pltpu.sync_copy(zacc_hbm.at[pl.ds(mrow, MPT)], acc_sh.at[pl.ds(mrow, MPT)])

  @pl.when(sid == 0)
  def _():
    pltpu.sync_copy(zacc_hbm.at[pl.ds(tail, M - tail)],
                    acc_sh.at[pl.ds(tail, M - tail)])

  zero16 = jnp.zeros((16,), jnp.int32)

  def zfill(i, carry):
    cnt_v[pl.ds(i * 16, 16)] = zero16
    return carry

  lax.fori_loop(0, (M + 16) // 16, zfill, None)
  plsc.subcore_barrier()

  onehot0 = jnp.where(lax.iota(jnp.int32, 16) == 0, 1, 0)

  def count_rows(idx_v):
    for j in range(CH // 16):
      v = idx_v[pl.ds(j * 16, 16)]
      for l in range(16):
        s = v[l]
        w = cnt_v[pl.ds(s, 16)]
        cnt_v[pl.ds(s, 16)] = w + onehot0

  def wait_chunk(q):
    pltpu.make_async_copy(ids_hbm.at[pl.ds(0, CH)],
                          idx_ring[q], sem_i[q]).wait()
    pltpu.make_async_copy(src_hbm.at[pl.ds(0, CH)],
                          buf_ring[q], sem_r[q]).wait()

  def process(q):
    wait_chunk(q)
    cp = pltpu.async_copy(buf_ring[q], acc_sh.at[idx_ring[q]], sem_s[q],
                          add=True)
    count_rows(idx_ring[q])
    cp.wait()

  def body(g, carry):
    for q in range(Q):
      process(q)
      nxt = Q * g + q + Q

      @pl.when(nxt < NCH)
      def _():
        fire_chunk(q, nxt)

    return carry

  lax.fori_loop(0, G, body, None)
  for q in range(NCH - Q * G):
    process(q)

  plsc.subcore_barrier()
  pltpu.sync_copy(acc_sh.at[pl.ds(mrow, MPT)],
                  sums_out.at[cid, pl.ds(mrow, MPT)])
  pltpu.sync_copy(cnt_v.at[pl.ds(0, M)], cnts_out.at[pl.ds(wid * M, M)])

  @pl.when(sid == 0)
  def _():
    pltpu.sync_copy(acc_sh.at[pl.ds(tail, M - tail)],
                    sums_out.at[cid, pl.ds(tail, M - tail)])


def _combine_body(s_ref, c_ref, o_ref):
  s = s_ref[0] + s_ref[1]
  c = jnp.sum(c_ref[...], axis=1, keepdims=True).astype(jnp.float32)
  o_ref[...] = s / jnp.maximum(c, 1.0)


_BM = 2000

_combine = pl.pallas_call(
    _combine_body,
    grid=(M // _BM,),
    in_specs=[
        pl.BlockSpec((NC, _BM, D), lambda i: (0, i, 0)),
        pl.BlockSpec((_BM, NW), lambda i: (i, 0)),
    ],
    out_specs=pl.BlockSpec((_BM, D), lambda i: (i, 0)),
    out_shape=jax.ShapeDtypeStruct((M, D), jnp.float32),
)


@jax.jit
def kernel(mesh_ids, sources):
  zacc = jnp.zeros((M, D), jnp.float32)
  sums, cnts = _segment_sum_sc(mesh_ids, sources, zacc)
  cnts_t = cnts.reshape(NW, M).T
  return _combine(sums, cnts_t)
